# baseline (speedup 1.0000x reference)
import functools

import jax
import jax.numpy as jnp
from jax import lax
from jax.experimental import pallas as pl
from jax.experimental.pallas import tpu as pltpu
from jax.experimental.pallas import tpu_sc as plsc

N = 10000
E = 320000
D = 128
H = 128
O = 32

NC = 2
NS = 16
L = 16

CHUNK = 128
CPT = 80
U = 8
QITERS = CPT // U
NCHUNKS_PAD = NC * NS * CPT + 8
E_PAD = NCHUNKS_PAD * CHUNK

ROWS_PER_TILE = 624
TAIL_ROWS = N - NS * ROWS_PER_TILE


def _make_agg(width):
  mesh = plsc.VectorSubcoreMesh(core_axis_name="c", subcore_axis_name="s")

  @functools.partial(
      pl.kernel,
      mesh=mesh,
      compiler_params=pltpu.CompilerParams(use_tc_tiling_on_sc=False),
      out_type=jax.ShapeDtypeStruct((NC, N, width), jnp.float32),
      scratch_types=[
          pltpu.VMEM((U, 2, CHUNK), jnp.int32),
          pltpu.VMEM((2, CHUNK, width), jnp.float32),
          pltpu.VMEM_SHARED((N + 8, width), jnp.float32),
      ] + [pltpu.SemaphoreType.DMA] * (U + 2),
  )
  def agg(x_hbm, sd_hbm, out_hbm, idx_v, rows_v, acc,
          si0, si1, si2, si3, si4, si5, si6, si7, sg0, sg1):
    sem_i = (si0, si1, si2, si3, si4, si5, si6, si7)
    sem_g = (sg0, sg1)
    cid = lax.axis_index("c")
    sid = lax.axis_index("s")
    base = (cid * NS + sid) * CPT

    def zrow(i, _):
      def zcol(j, _):
        rows_v[0, i, pl.ds(j * L, L)] = jnp.zeros((L,), jnp.float32)
        return 0
      return lax.fori_loop(0, width // L, zcol, 0)
    lax.fori_loop(0, CHUNK, zrow, 0)

    zbase = sid * ROWS_PER_TILE
    nfull = ROWS_PER_TILE // CHUNK
    rem = ROWS_PER_TILE - nfull * CHUNK
    for k in range(nfull):
      pltpu.sync_copy(rows_v.at[0], acc.at[pl.ds(zbase + k * CHUNK, CHUNK)])
    pltpu.sync_copy(rows_v.at[0, pl.ds(0, rem)],
                    acc.at[pl.ds(zbase + nfull * CHUNK, rem)])

    @pl.when(sid == NS - 1)
    def _():
      pltpu.sync_copy(rows_v.at[0, pl.ds(0, TAIL_ROWS)],
                      acc.at[pl.ds(NS * ROWS_PER_TILE, TAIL_ROWS)])
    plsc.subcore_barrier()

    def idx_copy(c, j):
      return pltpu.make_async_copy(sd_hbm.at[base + c], idx_v.at[j % U],
                                   sem_i[j % U])

    def gather(c, j):
      del c
      return pltpu.make_async_copy(x_hbm.at[idx_v.at[j % U, 0]],
                                   rows_v.at[j % 2], sem_g[j % 2])

    def scatter_prev(j):
      pltpu.sync_copy(rows_v.at[(j - 1) % 2],
                      acc.at[idx_v.at[(j - 1) % U, 1]], add=True)

    for j in range(U - 1):
      idx_copy(j, j).start()

    def qbody(q, _):
      c0 = q * U
      for j in range(U):
        c = c0 + j
        idx_copy(c, j).wait()
        gather(c, j).start()
        if j == 0:
          @pl.when(c >= 1)
          def _():
            gather(c - 1, j - 1).wait()
            scatter_prev(j)
        else:
          gather(c - 1, j - 1).wait()
          scatter_prev(j)
        idx_copy(c + U - 1, j - 1).start()
      return 0
    lax.fori_loop(0, QITERS, qbody, 0)

    gather(CPT - 1, U - 1).wait()
    scatter_prev(0)
    for j in range(U - 1):
      idx_copy(CPT + j, j).wait()
    plsc.subcore_barrier()

    pltpu.sync_copy(acc.at[pl.ds(zbase, ROWS_PER_TILE)],
                    out_hbm.at[cid, pl.ds(zbase, ROWS_PER_TILE)])

    @pl.when(sid == NS - 1)
    def _():
      pltpu.sync_copy(acc.at[pl.ds(NS * ROWS_PER_TILE, TAIL_ROWS)],
                      out_hbm.at[cid, pl.ds(NS * ROWS_PER_TILE, TAIL_ROWS)])

  return agg


_agg_d = _make_agg(D)
_agg_o = _make_agg(O)


def _tc1_body(agg_ref, xs_ref, w1r_ref, b1_ref, w1s_ref, bnw_ref, bnb_ref,
              w2r_ref, w2s_ref, z2_ref, z2root_ref):
  agg = agg_ref[0] + agg_ref[1]
  h = lax.dot_general(agg, w1r_ref[...], (((1,), (1,)), ((), ())),
                      preferred_element_type=jnp.float32)
  h += lax.dot_general(xs_ref[...], w1s_ref[...], (((1,), (1,)), ((), ())),
                       preferred_element_type=jnp.float32)
  h += b1_ref[...]
  m = jnp.mean(h, axis=0, keepdims=True)
  c = h - m
  v = jnp.mean(c * c, axis=0, keepdims=True)
  x1 = jnp.maximum(c * lax.rsqrt(v + 1e-5) * bnw_ref[...] + bnb_ref[...], 0.0)
  z2_ref[...] = lax.dot_general(x1, w2r_ref[...], (((1,), (1,)), ((), ())),
                                preferred_element_type=jnp.float32)
  z2root_ref[...] = lax.dot_general(x1, w2s_ref[...], (((1,), (1,)), ((), ())),
                                    preferred_element_type=jnp.float32)


_tc1 = pl.pallas_call(
    _tc1_body,
    out_shape=[jax.ShapeDtypeStruct((N, O), jnp.float32),
               jax.ShapeDtypeStruct((N, O), jnp.float32)],
)


def _tc2_body(agg_ref, z2root_ref, b2_ref, bnw_ref, bnb_ref, out_ref):
  h = agg_ref[0] + agg_ref[1] + z2root_ref[...] + b2_ref[...]
  m = jnp.mean(h, axis=0, keepdims=True)
  c = h - m
  v = jnp.mean(c * c, axis=0, keepdims=True)
  out_ref[...] = jnp.maximum(
      c * lax.rsqrt(v + 1e-5) * bnw_ref[...] + bnb_ref[...], 0.0)


_tc2 = pl.pallas_call(
    _tc2_body,
    out_shape=jax.ShapeDtypeStruct((N, O), jnp.float32),
)


def kernel(xs, edge_index, W1_rel, b1_rel, W1_root, bn1_w, bn1_b,
           W2_rel, b2_rel, W2_root, bn2_w, bn2_b):
  pad = E_PAD - E
  src_p = jnp.concatenate([edge_index[0], jnp.zeros((pad,), jnp.int32)])
  dst_p = jnp.concatenate([edge_index[1], jnp.full((pad,), N, jnp.int32)])
  sd = jnp.stack([src_p, dst_p]).reshape(2, NCHUNKS_PAD, CHUNK)
  sd = sd.transpose(1, 0, 2)

  agg1 = _agg_d(xs, sd)
  z2, z2root = _tc1(agg1, xs, W1_rel, b1_rel.reshape(1, H), W1_root,
                    bn1_w.reshape(1, H), bn1_b.reshape(1, H), W2_rel, W2_root)
  agg2 = _agg_o(z2, sd)
  return _tc2(agg2, z2root, b2_rel.reshape(1, O),
              bn2_w.reshape(1, O), bn2_b.reshape(1, O))
